# speedup vs baseline: 7.0761x; 7.0761x over previous
import functools

import jax
import jax.numpy as jnp
from jax.experimental import pallas as pl

N_U = 4096
N_I = 4096
DIM = 128
K = 10
NEG = -3.0e38


def _proj_norm_body(feat_ref, w_ref, b_ref, xn_ref):
    y = jnp.dot(feat_ref[...], w_ref[...], preferred_element_type=jnp.float32)
    y = y + b_ref[...]
    n = jnp.sqrt(jnp.sum(y * y, axis=1, keepdims=True))
    xn_ref[...] = y / n


def _proj_norm(feat, w, b):
    return pl.pallas_call(
        _proj_norm_body,
        out_shape=jax.ShapeDtypeStruct((N_I, DIM), jnp.float32),
    )(feat, w, b)


def _topk_body(xb_ref, xnt_ref, coeff_ref, idx_ref, d_ref):
    a = xb_ref[...]
    sim = jnp.dot(a, xnt_ref[...], preferred_element_type=jnp.float32)
    col = jax.lax.broadcasted_iota(jnp.int32, sim.shape, 1)
    work = sim
    vals = []
    idxs = []
    for _ in range(K):
        m = jnp.max(work, axis=1, keepdims=True)
        first = jnp.min(jnp.where(work == m, col, N_I), axis=1, keepdims=True)
        vals.append(m)
        idxs.append(first)
        work = jnp.where(col == first, NEG, work)
    v = jnp.concatenate(vals, axis=1)
    ix = jnp.concatenate(idxs, axis=1)
    rowsum = jnp.sum(v, axis=1, keepdims=True)
    dd = jnp.where(rowsum > 0.0,
                   jax.lax.rsqrt(jnp.where(rowsum > 0.0, rowsum, 1.0)), 0.0)
    coeff_ref[...] = v * dd
    idx_ref[...] = ix
    d_ref[...] = dd


def _topk(xn, xnt, blk=512):
    grid = (N_I // blk,)
    return pl.pallas_call(
        _topk_body,
        grid=grid,
        in_specs=[
            pl.BlockSpec((blk, DIM), lambda i: (i, 0)),
            pl.BlockSpec((DIM, N_I), lambda i: (0, 0)),
        ],
        out_specs=[
            pl.BlockSpec((blk, K), lambda i: (i, 0)),
            pl.BlockSpec((blk, K), lambda i: (i, 0)),
            pl.BlockSpec((blk, 1), lambda i: (i, 0)),
        ],
        out_shape=[
            jax.ShapeDtypeStruct((N_I, K), jnp.float32),
            jax.ShapeDtypeStruct((N_I, K), jnp.int32),
            jax.ShapeDtypeStruct((N_I, 1), jnp.float32),
        ],
    )(xn, xnt)


def _knn_prop_body(coeff_ref, idx_ref, d_ref, emb_ref, out_ref):
    demb = d_ref[...] * emb_ref[...]
    blk = coeff_ref.shape[0]
    col = jax.lax.broadcasted_iota(jnp.int32, (blk, N_I), 1)
    w = jnp.zeros((blk, N_I), jnp.float32)
    for j in range(K):
        w = w + jnp.where(col == idx_ref[:, j][:, None],
                          coeff_ref[:, j][:, None], 0.0)
    out_ref[...] = jnp.dot(w, demb, preferred_element_type=jnp.float32)


def _knn_prop(coeff, idx, d, emb, blk=512):
    grid = (N_I // blk,)
    return pl.pallas_call(
        _knn_prop_body,
        grid=grid,
        in_specs=[
            pl.BlockSpec((blk, K), lambda i: (i, 0)),
            pl.BlockSpec((blk, K), lambda i: (i, 0)),
            pl.BlockSpec((N_I, 1), lambda i: (0, 0)),
            pl.BlockSpec((N_I, DIM), lambda i: (0, 0)),
        ],
        out_specs=pl.BlockSpec((blk, DIM), lambda i: (i, 0)),
        out_shape=jax.ShapeDtypeStruct((N_I, DIM), jnp.float32),
    )(coeff, idx, d, emb)


def _blend_mm_body(a_ref, x_ref, k_ref, out_ref):
    acc = jnp.dot(a_ref[...], x_ref[...], preferred_element_type=jnp.float32)
    out_ref[...] = 0.9 * acc + (1.0 - 0.9) * k_ref[...]


def _blend_mm(a, x, knn, blk=512):
    grid = (N_I // blk,)
    return pl.pallas_call(
        _blend_mm_body,
        grid=grid,
        in_specs=[
            pl.BlockSpec((blk, N_I), lambda i: (i, 0)),
            pl.BlockSpec((N_I, DIM), lambda i: (0, 0)),
            pl.BlockSpec((blk, DIM), lambda i: (i, 0)),
        ],
        out_specs=pl.BlockSpec((blk, DIM), lambda i: (i, 0)),
        out_shape=jax.ShapeDtypeStruct((N_I, DIM), jnp.float32),
    )(a, x, knn)


def _mm_body(a_ref, x_ref, out_ref):
    out_ref[...] = jnp.dot(a_ref[...], x_ref[...],
                           preferred_element_type=jnp.float32)


def _mm(a, x, blk=512):
    n = a.shape[0]
    return pl.pallas_call(
        _mm_body,
        grid=(n // blk,),
        in_specs=[
            pl.BlockSpec((blk, n), lambda i: (i, 0)),
            pl.BlockSpec((n, DIM), lambda i: (0, 0)),
        ],
        out_specs=pl.BlockSpec((blk, DIM), lambda i: (i, 0)),
        out_shape=jax.ShapeDtypeStruct((n, DIM), jnp.float32),
    )(a, x)


def _mm_mean_body(a_ref, x_ref, e0_ref, e1_ref, out_ref):
    acc = jnp.dot(a_ref[...], x_ref[...], preferred_element_type=jnp.float32)
    out_ref[...] = (e0_ref[...] + e1_ref[...] + acc) / 3.0


def _mm_mean(a, x, e0, blk=512):
    n = a.shape[0]
    return pl.pallas_call(
        _mm_mean_body,
        grid=(n // blk,),
        in_specs=[
            pl.BlockSpec((blk, n), lambda i: (i, 0)),
            pl.BlockSpec((n, DIM), lambda i: (0, 0)),
            pl.BlockSpec((blk, DIM), lambda i: (i, 0)),
            pl.BlockSpec((blk, DIM), lambda i: (i, 0)),
        ],
        out_specs=pl.BlockSpec((blk, DIM), lambda i: (i, 0)),
        out_shape=jax.ShapeDtypeStruct((n, DIM), jnp.float32),
    )(a, x, e0, x)


def _fusion_body(img_ref, txt_ref, wq1_ref, bq1_ref, wq2_ref, mi_ref,
                 h_ref, ig_ref):
    img = img_ref[...]
    txt = txt_ref[...]
    w1 = wq1_ref[...]
    b1 = bq1_ref[...]
    w2 = wq2_ref[...]
    qi = jnp.dot(jnp.tanh(jnp.dot(img, w1, preferred_element_type=jnp.float32)
                          + b1), w2, preferred_element_type=jnp.float32)
    qt = jnp.dot(jnp.tanh(jnp.dot(txt, w1, preferred_element_type=jnp.float32)
                          + b1), w2, preferred_element_type=jnp.float32)
    m = jnp.maximum(qi, qt)
    ei = jnp.exp(qi - m)
    et = jnp.exp(qt - m)
    s = ei + et
    h = (ei / s) * img + (et / s) * txt
    h_ref[...] = h
    ig_ref[...] = mi_ref[...] + h


def _fusion(img_e, txt_e, wq1, bq1, wq2, mean_items):
    return pl.pallas_call(
        _fusion_body,
        out_shape=[
            jax.ShapeDtypeStruct((N_I, DIM), jnp.float32),
            jax.ShapeDtypeStruct((N_I, DIM), jnp.float32),
        ],
    )(img_e, txt_e, wq1, bq1, wq2, mean_items)


def kernel(adj, feat_visual, feat_text, user_emb, item_emb, W_img, b_img,
           W_txt, b_txt, Wq1, bq1, Wq2, image_original_adj, text_original_adj):
    xn_i = _proj_norm(feat_visual, W_img, b_img.reshape(1, DIM))
    xn_t = _proj_norm(feat_text, W_txt, b_txt.reshape(1, DIM))
    coeff_i, idx_i, d_i = _topk(xn_i, xn_i.T)
    coeff_t, idx_t, d_t = _topk(xn_t, xn_t.T)
    knn_i = _knn_prop(coeff_i, idx_i, d_i, item_emb)
    knn_t = _knn_prop(coeff_t, idx_t, d_t, item_emb)
    img_e = _blend_mm(image_original_adj, item_emb, knn_i)
    txt_e = _blend_mm(text_original_adj, item_emb, knn_t)
    ego0 = jnp.concatenate([user_emb, item_emb], axis=0)
    ego1 = _mm(adj, ego0)
    mean_emb = _mm_mean(adj, ego1, ego0)
    h, ig = _fusion(img_e, txt_e, Wq1, bq1.reshape(1, DIM), Wq2,
                    mean_emb[N_U:])
    return (mean_emb[:N_U], ig, img_e, txt_e, h)


# speedup vs baseline: 7.3726x; 1.0419x over previous
import functools

import jax
import jax.numpy as jnp
from jax import lax
from jax.experimental import pallas as pl
from jax.experimental.pallas import tpu as pltpu
from jax.experimental.pallas import tpu_sc as plsc

N_U = 4096
N_I = 4096
DIM = 128
K = 10
NEG = -3.0e38


def _proj_norm_body(feat_ref, w_ref, b_ref, xn_ref):
    y = jnp.dot(feat_ref[...], w_ref[...], preferred_element_type=jnp.float32)
    y = y + b_ref[...]
    n = jnp.sqrt(jnp.sum(y * y, axis=1, keepdims=True))
    xn_ref[...] = y / n


def _proj_norm(feat, w, b):
    return pl.pallas_call(
        _proj_norm_body,
        out_shape=jax.ShapeDtypeStruct((N_I, DIM), jnp.float32),
    )(feat, w, b)


def _topk_body(xb_ref, xnt_ref, wrep_ref, idx_ref, d_ref):
    a = xb_ref[...]
    sim = jnp.dot(a, xnt_ref[...], preferred_element_type=jnp.float32)
    col = jax.lax.broadcasted_iota(jnp.int32, sim.shape, 1)
    work = sim
    vals = []
    idxs = []
    for _ in range(K):
        m = jnp.max(work, axis=1, keepdims=True)
        first = jnp.min(jnp.where(work == m, col, N_I), axis=1, keepdims=True)
        vals.append(m)
        idxs.append(first)
        work = jnp.where(col == first, NEG, work)
    v = jnp.concatenate(vals, axis=1)
    ix = jnp.concatenate(idxs, axis=1)
    rowsum = jnp.sum(v, axis=1, keepdims=True)
    dd = jnp.where(rowsum > 0.0,
                   jax.lax.rsqrt(jnp.where(rowsum > 0.0, rowsum, 1.0)), 0.0)
    coeff = v * dd
    wrep_ref[...] = jnp.broadcast_to(coeff[:, :, None],
                                     coeff.shape + (16,)).reshape(
                                         coeff.shape[0], K * 16)
    idx_ref[...] = ix
    d_ref[...] = dd


def _topk(xn, xnt, blk=512):
    grid = (N_I // blk,)
    return pl.pallas_call(
        _topk_body,
        grid=grid,
        in_specs=[
            pl.BlockSpec((blk, DIM), lambda i: (i, 0)),
            pl.BlockSpec((DIM, N_I), lambda i: (0, 0)),
        ],
        out_specs=[
            pl.BlockSpec((blk, K * 16), lambda i: (i, 0)),
            pl.BlockSpec((blk, K), lambda i: (i, 0)),
            pl.BlockSpec((blk, 1), lambda i: (i, 0)),
        ],
        out_shape=[
            jax.ShapeDtypeStruct((N_I, K * 16), jnp.float32),
            jax.ShapeDtypeStruct((N_I, K), jnp.int32),
            jax.ShapeDtypeStruct((N_I, 1), jnp.float32),
        ],
    )(xn, xnt)


def _scale_emb_body(di_ref, dt_ref, emb_ref, oi_ref, ot_ref):
    e = emb_ref[...]
    oi_ref[...] = di_ref[...] * e
    ot_ref[...] = dt_ref[...] * e


def _scale_emb(d_i, d_t, emb):
    return pl.pallas_call(
        _scale_emb_body,
        out_shape=[
            jax.ShapeDtypeStruct((N_I, DIM), jnp.float32),
            jax.ShapeDtypeStruct((N_I, DIM), jnp.float32),
        ],
    )(d_i, d_t, emb)


def _knn_prop_body(coeff_ref, idx_ref, d_ref, emb_ref, out_ref):
    demb = d_ref[...] * emb_ref[...]
    blk = coeff_ref.shape[0]
    col = jax.lax.broadcasted_iota(jnp.int32, (blk, N_I), 1)
    w = jnp.zeros((blk, N_I), jnp.float32)
    for j in range(K):
        w = w + jnp.where(col == idx_ref[:, j][:, None],
                          coeff_ref[:, j][:, None], 0.0)
    out_ref[...] = jnp.dot(w, demb, preferred_element_type=jnp.float32)


def _knn_prop(coeff, idx, d, emb, blk=512):
    grid = (N_I // blk,)
    return pl.pallas_call(
        _knn_prop_body,
        grid=grid,
        in_specs=[
            pl.BlockSpec((blk, K), lambda i: (i, 0)),
            pl.BlockSpec((blk, K), lambda i: (i, 0)),
            pl.BlockSpec((N_I, 1), lambda i: (0, 0)),
            pl.BlockSpec((N_I, DIM), lambda i: (0, 0)),
        ],
        out_specs=pl.BlockSpec((blk, DIM), lambda i: (i, 0)),
        out_shape=jax.ShapeDtypeStruct((N_I, DIM), jnp.float32),
    )(coeff, idx, d, emb)


_SC_ROWS = 128
_SC_CHUNK = 32
_SC_EDGE = _SC_CHUNK * K
_SC_GB = 64


def _sc_knn_modality(idx_hbm, wrep_hbm, demb_hbm, out_hbm,
                     idx_v, w_v, rows_v, out_v, sem, wid):
    row0 = wid * _SC_ROWS

    def do_chunk(c, _):
        rbase = row0 + c * _SC_CHUNK
        ebase = rbase * K
        pltpu.sync_copy(idx_hbm.at[pl.ds(ebase, _SC_EDGE)], idx_v)
        pltpu.sync_copy(wrep_hbm.at[pl.ds(ebase, _SC_EDGE)], w_v)
        copies = [
            pltpu.async_copy(demb_hbm.at[idx_v.at[pl.ds(b * _SC_GB, _SC_GB)]],
                             rows_v.at[pl.ds(b * _SC_GB, _SC_GB)], sem)
            for b in range(_SC_EDGE // _SC_GB)
        ]
        for cp in copies:
            cp.wait()

        def row_acc(r, _):
            e0 = r * K
            acc = [jnp.zeros((16,), jnp.float32) for _ in range(DIM // 16)]
            for j in range(K):
                wj = w_v[e0 + j, pl.ds(0, 16)]
                for g in range(DIM // 16):
                    acc[g] = acc[g] + wj * rows_v[e0 + j, pl.ds(g * 16, 16)]
            for g in range(DIM // 16):
                out_v[r, pl.ds(g * 16, 16)] = acc[g]
            return ()

        lax.fori_loop(0, _SC_CHUNK, row_acc, ())
        pltpu.sync_copy(out_v, out_hbm.at[pl.ds(rbase, _SC_CHUNK)])
        return ()

    lax.fori_loop(0, _SC_ROWS // _SC_CHUNK, do_chunk, ())


def _sc_knn_body(idx_i, wrep_i, demb_i, idx_t, wrep_t, demb_t,
                 out_i, out_t, idx_v, w_v, rows_v, out_v, sem):
    wid = lax.axis_index("s") * 2 + lax.axis_index("c")
    _sc_knn_modality(idx_i, wrep_i, demb_i, out_i,
                     idx_v, w_v, rows_v, out_v, sem, wid)
    _sc_knn_modality(idx_t, wrep_t, demb_t, out_t,
                     idx_v, w_v, rows_v, out_v, sem, wid)


def _sc_knn(idx_i, wrep_i, demb_i, idx_t, wrep_t, demb_t):
    f32 = jnp.float32
    return pl.kernel(
        _sc_knn_body,
        mesh=plsc.VectorSubcoreMesh(core_axis_name="c", subcore_axis_name="s"),
        out_type=[
            jax.ShapeDtypeStruct((N_I, DIM), f32),
            jax.ShapeDtypeStruct((N_I, DIM), f32),
        ],
        scratch_types=[
            pltpu.VMEM((_SC_EDGE,), jnp.int32),
            pltpu.VMEM((_SC_EDGE, 16), f32),
            pltpu.VMEM((_SC_EDGE, DIM), f32),
            pltpu.VMEM((_SC_CHUNK, DIM), f32),
            pltpu.SemaphoreType.DMA,
        ],
    )(idx_i, wrep_i, demb_i, idx_t, wrep_t, demb_t)


def _blend_mm_body(a_ref, x_ref, k_ref, out_ref):
    acc = jnp.dot(a_ref[...], x_ref[...], preferred_element_type=jnp.float32)
    out_ref[...] = 0.9 * acc + (1.0 - 0.9) * k_ref[...]


def _blend_mm(a, x, knn, blk=512):
    grid = (N_I // blk,)
    return pl.pallas_call(
        _blend_mm_body,
        grid=grid,
        in_specs=[
            pl.BlockSpec((blk, N_I), lambda i: (i, 0)),
            pl.BlockSpec((N_I, DIM), lambda i: (0, 0)),
            pl.BlockSpec((blk, DIM), lambda i: (i, 0)),
        ],
        out_specs=pl.BlockSpec((blk, DIM), lambda i: (i, 0)),
        out_shape=jax.ShapeDtypeStruct((N_I, DIM), jnp.float32),
    )(a, x, knn)


def _mm_body(a_ref, x_ref, out_ref):
    out_ref[...] = jnp.dot(a_ref[...], x_ref[...],
                           preferred_element_type=jnp.float32)


def _mm(a, x, blk=512):
    n = a.shape[0]
    return pl.pallas_call(
        _mm_body,
        grid=(n // blk,),
        in_specs=[
            pl.BlockSpec((blk, n), lambda i: (i, 0)),
            pl.BlockSpec((n, DIM), lambda i: (0, 0)),
        ],
        out_specs=pl.BlockSpec((blk, DIM), lambda i: (i, 0)),
        out_shape=jax.ShapeDtypeStruct((n, DIM), jnp.float32),
    )(a, x)


def _mm_mean_body(a_ref, x_ref, e0_ref, e1_ref, out_ref):
    acc = jnp.dot(a_ref[...], x_ref[...], preferred_element_type=jnp.float32)
    out_ref[...] = (e0_ref[...] + e1_ref[...] + acc) / 3.0


def _mm_mean(a, x, e0, blk=512):
    n = a.shape[0]
    return pl.pallas_call(
        _mm_mean_body,
        grid=(n // blk,),
        in_specs=[
            pl.BlockSpec((blk, n), lambda i: (i, 0)),
            pl.BlockSpec((n, DIM), lambda i: (0, 0)),
            pl.BlockSpec((blk, DIM), lambda i: (i, 0)),
            pl.BlockSpec((blk, DIM), lambda i: (i, 0)),
        ],
        out_specs=pl.BlockSpec((blk, DIM), lambda i: (i, 0)),
        out_shape=jax.ShapeDtypeStruct((n, DIM), jnp.float32),
    )(a, x, e0, x)


def _fusion_body(img_ref, txt_ref, wq1_ref, bq1_ref, wq2_ref, mi_ref,
                 h_ref, ig_ref):
    img = img_ref[...]
    txt = txt_ref[...]
    w1 = wq1_ref[...]
    b1 = bq1_ref[...]
    w2 = wq2_ref[...]
    qi = jnp.dot(jnp.tanh(jnp.dot(img, w1, preferred_element_type=jnp.float32)
                          + b1), w2, preferred_element_type=jnp.float32)
    qt = jnp.dot(jnp.tanh(jnp.dot(txt, w1, preferred_element_type=jnp.float32)
                          + b1), w2, preferred_element_type=jnp.float32)
    m = jnp.maximum(qi, qt)
    ei = jnp.exp(qi - m)
    et = jnp.exp(qt - m)
    s = ei + et
    h = (ei / s) * img + (et / s) * txt
    h_ref[...] = h
    ig_ref[...] = mi_ref[...] + h


def _fusion(img_e, txt_e, wq1, bq1, wq2, mean_items):
    return pl.pallas_call(
        _fusion_body,
        out_shape=[
            jax.ShapeDtypeStruct((N_I, DIM), jnp.float32),
            jax.ShapeDtypeStruct((N_I, DIM), jnp.float32),
        ],
    )(img_e, txt_e, wq1, bq1, wq2, mean_items)


def kernel(adj, feat_visual, feat_text, user_emb, item_emb, W_img, b_img,
           W_txt, b_txt, Wq1, bq1, Wq2, image_original_adj, text_original_adj):
    xn_i = _proj_norm(feat_visual, W_img, b_img.reshape(1, DIM))
    xn_t = _proj_norm(feat_text, W_txt, b_txt.reshape(1, DIM))
    wrep_i, idx_i, d_i = _topk(xn_i, xn_i.T)
    wrep_t, idx_t, d_t = _topk(xn_t, xn_t.T)
    demb_i, demb_t = _scale_emb(d_i, d_t, item_emb)
    knn_i, knn_t = _sc_knn(idx_i.reshape(-1), wrep_i.reshape(N_I * K, 16),
                           demb_i, idx_t.reshape(-1),
                           wrep_t.reshape(N_I * K, 16), demb_t)
    img_e = _blend_mm(image_original_adj, item_emb, knn_i)
    txt_e = _blend_mm(text_original_adj, item_emb, knn_t)
    ego0 = jnp.concatenate([user_emb, item_emb], axis=0)
    ego1 = _mm(adj, ego0)
    mean_emb = _mm_mean(adj, ego1, ego0)
    h, ig = _fusion(img_e, txt_e, Wq1, bq1.reshape(1, DIM), Wq2,
                    mean_emb[N_U:])
    return (mean_emb[:N_U], ig, img_e, txt_e, h)


# speedup vs baseline: 8.1454x; 1.1048x over previous
import functools

import jax
import jax.numpy as jnp
from jax import lax
from jax.experimental import pallas as pl
from jax.experimental.pallas import tpu as pltpu
from jax.experimental.pallas import tpu_sc as plsc

N_U = 4096
N_I = 4096
DIM = 128
K = 10
NEG = -3.0e38


def _proj_norm_body(feat_ref, w_ref, b_ref, xn_ref):
    y = jnp.dot(feat_ref[...], w_ref[...], preferred_element_type=jnp.float32)
    y = y + b_ref[...]
    n = jnp.sqrt(jnp.sum(y * y, axis=1, keepdims=True))
    xn_ref[...] = y / n


def _proj_norm(feat, w, b):
    return pl.pallas_call(
        _proj_norm_body,
        out_shape=jax.ShapeDtypeStruct((N_I, DIM), jnp.float32),
    )(feat, w, b)


def _topk_compute(xb_ref, xnt_ref, wrep_ref, idx_ref, d_ref):
    a = xb_ref[...]
    sim = jnp.dot(a, xnt_ref[...], preferred_element_type=jnp.float32)
    col = jax.lax.broadcasted_iota(jnp.int32, sim.shape, 1)
    work = sim
    vals = []
    idxs = []
    for _ in range(K):
        m = jnp.max(work, axis=1, keepdims=True)
        first = jnp.min(jnp.where(work == m, col, N_I), axis=1, keepdims=True)
        vals.append(m)
        idxs.append(first)
        work = jnp.where(col == first, NEG, work)
    v = jnp.concatenate(vals, axis=1)
    ix = jnp.concatenate(idxs, axis=1)
    rowsum = jnp.sum(v, axis=1, keepdims=True)
    dd = jnp.where(rowsum > 0.0,
                   jax.lax.rsqrt(jnp.where(rowsum > 0.0, rowsum, 1.0)), 0.0)
    coeff = v * dd
    wrep_ref[...] = jnp.broadcast_to(coeff[:, :, None],
                                     coeff.shape + (16,)).reshape(
                                         coeff.shape[0], K * 16)
    idx_ref[...] = ix
    d_ref[...] = dd


def _l1_body(adj_ref, ego_ref, xbi_ref, xnti_ref, xbt_ref, xntt_ref,
             ego1_ref, wrepi_ref, idxi_ref, di_ref,
             wrept_ref, idxt_ref, dt_ref):
    ego1_ref[...] = jnp.dot(adj_ref[...], ego_ref[...],
                            preferred_element_type=jnp.float32)
    _topk_compute(xbi_ref, xnti_ref, wrepi_ref, idxi_ref, di_ref)
    _topk_compute(xbt_ref, xntt_ref, wrept_ref, idxt_ref, dt_ref)


def _l1(adj, ego0, xn_i, xnt_i, xn_t, xnt_t, blk=256):
    n = adj.shape[0]
    grid = (n // blk,)
    tblk = N_I // (n // blk)
    f32 = jnp.float32
    return pl.pallas_call(
        _l1_body,
        grid=grid,
        in_specs=[
            pl.BlockSpec((blk, n), lambda i: (i, 0)),
            pl.BlockSpec((n, DIM), lambda i: (0, 0)),
            pl.BlockSpec((tblk, DIM), lambda i: (i, 0)),
            pl.BlockSpec((DIM, N_I), lambda i: (0, 0)),
            pl.BlockSpec((tblk, DIM), lambda i: (i, 0)),
            pl.BlockSpec((DIM, N_I), lambda i: (0, 0)),
        ],
        out_specs=[
            pl.BlockSpec((blk, DIM), lambda i: (i, 0)),
            pl.BlockSpec((tblk, K * 16), lambda i: (i, 0)),
            pl.BlockSpec((tblk, K), lambda i: (i, 0)),
            pl.BlockSpec((tblk, 1), lambda i: (i, 0)),
            pl.BlockSpec((tblk, K * 16), lambda i: (i, 0)),
            pl.BlockSpec((tblk, K), lambda i: (i, 0)),
            pl.BlockSpec((tblk, 1), lambda i: (i, 0)),
        ],
        out_shape=[
            jax.ShapeDtypeStruct((n, DIM), f32),
            jax.ShapeDtypeStruct((N_I, K * 16), f32),
            jax.ShapeDtypeStruct((N_I, K), jnp.int32),
            jax.ShapeDtypeStruct((N_I, 1), f32),
            jax.ShapeDtypeStruct((N_I, K * 16), f32),
            jax.ShapeDtypeStruct((N_I, K), jnp.int32),
            jax.ShapeDtypeStruct((N_I, 1), f32),
        ],
    )(adj, ego0, xn_i, xnt_i, xn_t, xnt_t)


def _scale_emb_body(di_ref, dt_ref, emb_ref, oi_ref, ot_ref):
    e = emb_ref[...]
    oi_ref[...] = di_ref[...] * e
    ot_ref[...] = dt_ref[...] * e


def _scale_emb(d_i, d_t, emb):
    return pl.pallas_call(
        _scale_emb_body,
        out_shape=[
            jax.ShapeDtypeStruct((N_I, DIM), jnp.float32),
            jax.ShapeDtypeStruct((N_I, DIM), jnp.float32),
        ],
    )(d_i, d_t, emb)


_SC_ROWS = 128
_SC_CHUNK = 32
_SC_EDGE = _SC_CHUNK * K
_SC_GB = 64


def _sc_knn_modality(idx_hbm, wrep_hbm, demb_hbm, out_hbm,
                     idx_v, w_v, rows_v, out_v, sem, wid):
    row0 = wid * _SC_ROWS

    def do_chunk(c, _):
        rbase = row0 + c * _SC_CHUNK
        ebase = rbase * K
        pltpu.sync_copy(idx_hbm.at[pl.ds(ebase, _SC_EDGE)], idx_v)
        pltpu.sync_copy(wrep_hbm.at[pl.ds(ebase, _SC_EDGE)], w_v)
        copies = [
            pltpu.async_copy(demb_hbm.at[idx_v.at[pl.ds(b * _SC_GB, _SC_GB)]],
                             rows_v.at[pl.ds(b * _SC_GB, _SC_GB)], sem)
            for b in range(_SC_EDGE // _SC_GB)
        ]
        for cp in copies:
            cp.wait()

        def row_acc(r, _):
            e0 = r * K
            acc = [jnp.zeros((16,), jnp.float32) for _ in range(DIM // 16)]
            for j in range(K):
                wj = w_v[e0 + j, pl.ds(0, 16)]
                for g in range(DIM // 16):
                    acc[g] = acc[g] + wj * rows_v[e0 + j, pl.ds(g * 16, 16)]
            for g in range(DIM // 16):
                out_v[r, pl.ds(g * 16, 16)] = acc[g]
            return ()

        lax.fori_loop(0, _SC_CHUNK, row_acc, ())
        pltpu.sync_copy(out_v, out_hbm.at[pl.ds(rbase, _SC_CHUNK)])
        return ()

    lax.fori_loop(0, _SC_ROWS // _SC_CHUNK, do_chunk, ())


def _sc_knn_body(idx_i, wrep_i, demb_i, idx_t, wrep_t, demb_t,
                 out_i, out_t, idx_v, w_v, rows_v, out_v, sem):
    wid = lax.axis_index("s") * 2 + lax.axis_index("c")
    _sc_knn_modality(idx_i, wrep_i, demb_i, out_i,
                     idx_v, w_v, rows_v, out_v, sem, wid)
    _sc_knn_modality(idx_t, wrep_t, demb_t, out_t,
                     idx_v, w_v, rows_v, out_v, sem, wid)


def _sc_knn(idx_i, wrep_i, demb_i, idx_t, wrep_t, demb_t):
    f32 = jnp.float32
    return pl.kernel(
        _sc_knn_body,
        mesh=plsc.VectorSubcoreMesh(core_axis_name="c", subcore_axis_name="s"),
        out_type=[
            jax.ShapeDtypeStruct((N_I, DIM), f32),
            jax.ShapeDtypeStruct((N_I, DIM), f32),
        ],
        scratch_types=[
            pltpu.VMEM((_SC_EDGE,), jnp.int32),
            pltpu.VMEM((_SC_EDGE, 16), f32),
            pltpu.VMEM((_SC_EDGE, DIM), f32),
            pltpu.VMEM((_SC_CHUNK, DIM), f32),
            pltpu.SemaphoreType.DMA,
        ],
    )(idx_i, wrep_i, demb_i, idx_t, wrep_t, demb_t)


def _l2_body(adj_ref, ego1f_ref, e0_ref, e1_ref, oi_ref, ot_ref, emb_ref,
             mean_ref, mmi_ref, mmt_ref):
    acc = jnp.dot(adj_ref[...], ego1f_ref[...],
                  preferred_element_type=jnp.float32)
    mean_ref[...] = (e0_ref[...] + e1_ref[...] + acc) / 3.0
    emb = emb_ref[...]
    mmi_ref[...] = jnp.dot(oi_ref[...], emb,
                           preferred_element_type=jnp.float32)
    mmt_ref[...] = jnp.dot(ot_ref[...], emb,
                           preferred_element_type=jnp.float32)


def _l2(adj, ego1, ego0, orig_i, orig_t, emb, blk=256):
    n = adj.shape[0]
    grid = (n // blk,)
    oblk = N_I // (n // blk)
    f32 = jnp.float32
    return pl.pallas_call(
        _l2_body,
        grid=grid,
        in_specs=[
            pl.BlockSpec((blk, n), lambda i: (i, 0)),
            pl.BlockSpec((n, DIM), lambda i: (0, 0)),
            pl.BlockSpec((blk, DIM), lambda i: (i, 0)),
            pl.BlockSpec((blk, DIM), lambda i: (i, 0)),
            pl.BlockSpec((oblk, N_I), lambda i: (i, 0)),
            pl.BlockSpec((oblk, N_I), lambda i: (i, 0)),
            pl.BlockSpec((N_I, DIM), lambda i: (0, 0)),
        ],
        out_specs=[
            pl.BlockSpec((blk, DIM), lambda i: (i, 0)),
            pl.BlockSpec((oblk, DIM), lambda i: (i, 0)),
            pl.BlockSpec((oblk, DIM), lambda i: (i, 0)),
        ],
        out_shape=[
            jax.ShapeDtypeStruct((n, DIM), f32),
            jax.ShapeDtypeStruct((N_I, DIM), f32),
            jax.ShapeDtypeStruct((N_I, DIM), f32),
        ],
    )(adj, ego1, ego0, ego1, orig_i, orig_t, emb)


def _fusion_body(mmi_ref, mmt_ref, ki_ref, kt_ref, wq1_ref, bq1_ref,
                 wq2_ref, mi_ref, img_ref, txt_ref, h_ref, ig_ref):
    img = 0.9 * mmi_ref[...] + (1.0 - 0.9) * ki_ref[...]
    txt = 0.9 * mmt_ref[...] + (1.0 - 0.9) * kt_ref[...]
    w1 = wq1_ref[...]
    b1 = bq1_ref[...]
    w2 = wq2_ref[...]
    qi = jnp.dot(jnp.tanh(jnp.dot(img, w1, preferred_element_type=jnp.float32)
                          + b1), w2, preferred_element_type=jnp.float32)
    qt = jnp.dot(jnp.tanh(jnp.dot(txt, w1, preferred_element_type=jnp.float32)
                          + b1), w2, preferred_element_type=jnp.float32)
    m = jnp.maximum(qi, qt)
    ei = jnp.exp(qi - m)
    et = jnp.exp(qt - m)
    s = ei + et
    h = (ei / s) * img + (et / s) * txt
    img_ref[...] = img
    txt_ref[...] = txt
    h_ref[...] = h
    ig_ref[...] = mi_ref[...] + h


def _fusion(mm_img, mm_txt, knn_i, knn_t, wq1, bq1, wq2, mean_items):
    f32 = jnp.float32
    return pl.pallas_call(
        _fusion_body,
        out_shape=[
            jax.ShapeDtypeStruct((N_I, DIM), f32),
            jax.ShapeDtypeStruct((N_I, DIM), f32),
            jax.ShapeDtypeStruct((N_I, DIM), f32),
            jax.ShapeDtypeStruct((N_I, DIM), f32),
        ],
    )(mm_img, mm_txt, knn_i, knn_t, wq1, bq1, wq2, mean_items)


def kernel(adj, feat_visual, feat_text, user_emb, item_emb, W_img, b_img,
           W_txt, b_txt, Wq1, bq1, Wq2, image_original_adj, text_original_adj):
    xn_i = _proj_norm(feat_visual, W_img, b_img.reshape(1, DIM))
    xn_t = _proj_norm(feat_text, W_txt, b_txt.reshape(1, DIM))
    ego0 = jnp.concatenate([user_emb, item_emb], axis=0)
    (ego1, wrep_i, idx_i, d_i,
     wrep_t, idx_t, d_t) = _l1(adj, ego0, xn_i, xn_i.T, xn_t, xn_t.T)
    demb_i, demb_t = _scale_emb(d_i, d_t, item_emb)
    knn_i, knn_t = _sc_knn(idx_i.reshape(-1), wrep_i.reshape(N_I * K, 16),
                           demb_i, idx_t.reshape(-1),
                           wrep_t.reshape(N_I * K, 16), demb_t)
    mean_emb, mm_img, mm_txt = _l2(adj, ego1, ego0, image_original_adj,
                                   text_original_adj, item_emb)
    img_e, txt_e, h, ig = _fusion(mm_img, mm_txt, knn_i, knn_t, Wq1,
                                  bq1.reshape(1, DIM), Wq2, mean_emb[N_U:])
    return (mean_emb[:N_U], ig, img_e, txt_e, h)


# speedup vs baseline: 8.6451x; 1.0614x over previous
import functools

import jax
import jax.numpy as jnp
from jax import lax
from jax.experimental import pallas as pl
from jax.experimental.pallas import tpu as pltpu
from jax.experimental.pallas import tpu_sc as plsc

N_U = 4096
N_I = 4096
DIM = 128
K = 10
NEG = -3.0e38


def _proj_norm_body(feat_ref, w_ref, b_ref, xn_ref):
    y = jnp.dot(feat_ref[...], w_ref[...], preferred_element_type=jnp.float32)
    y = y + b_ref[...]
    n = jnp.sqrt(jnp.sum(y * y, axis=1, keepdims=True))
    xn_ref[...] = y / n


def _proj_norm(feat, w, b):
    return pl.pallas_call(
        _proj_norm_body,
        out_shape=jax.ShapeDtypeStruct((N_I, DIM), jnp.float32),
    )(feat, w, b)


def _topk_compute(xb_ref, xnt_ref, wrep_ref, idx_ref, d_ref):
    a = xb_ref[...]
    sim = jnp.dot(a, xnt_ref[...], preferred_element_type=jnp.float32)
    col = jax.lax.broadcasted_iota(jnp.int32, sim.shape, 1)
    work = sim
    vals = []
    idxs = []
    for _ in range(K):
        m = jnp.max(work, axis=1, keepdims=True)
        ismax = work >= m
        idxs.append(jnp.min(jnp.where(ismax, col, N_I), axis=1,
                            keepdims=True))
        vals.append(m)
        work = jnp.where(ismax, NEG, work)
    v = jnp.concatenate(vals, axis=1)
    ix = jnp.concatenate(idxs, axis=1)
    rowsum = jnp.sum(v, axis=1, keepdims=True)
    dd = jnp.where(rowsum > 0.0,
                   jax.lax.rsqrt(jnp.where(rowsum > 0.0, rowsum, 1.0)), 0.0)
    coeff = v * dd
    wrep_ref[...] = jnp.broadcast_to(coeff[:, :, None],
                                     coeff.shape + (16,)).reshape(
                                         coeff.shape[0], K * 16)
    idx_ref[...] = ix
    d_ref[...] = dd


def _l1_body(adj_ref, ego_ref, xbi_ref, xnti_ref, xbt_ref, xntt_ref,
             ego1_ref, wrepi_ref, idxi_ref, di_ref,
             wrept_ref, idxt_ref, dt_ref):
    ego1_ref[...] = jnp.dot(adj_ref[...], ego_ref[...],
                            preferred_element_type=jnp.float32)
    _topk_compute(xbi_ref, xnti_ref, wrepi_ref, idxi_ref, di_ref)
    _topk_compute(xbt_ref, xntt_ref, wrept_ref, idxt_ref, dt_ref)


def _l1(adj, ego0, xn_i, xnt_i, xn_t, xnt_t, blk=256):
    n = adj.shape[0]
    grid = (n // blk,)
    tblk = N_I // (n // blk)
    f32 = jnp.float32
    return pl.pallas_call(
        _l1_body,
        grid=grid,
        in_specs=[
            pl.BlockSpec((blk, n), lambda i: (i, 0)),
            pl.BlockSpec((n, DIM), lambda i: (0, 0)),
            pl.BlockSpec((tblk, DIM), lambda i: (i, 0)),
            pl.BlockSpec((DIM, N_I), lambda i: (0, 0)),
            pl.BlockSpec((tblk, DIM), lambda i: (i, 0)),
            pl.BlockSpec((DIM, N_I), lambda i: (0, 0)),
        ],
        out_specs=[
            pl.BlockSpec((blk, DIM), lambda i: (i, 0)),
            pl.BlockSpec((tblk, K * 16), lambda i: (i, 0)),
            pl.BlockSpec((tblk, K), lambda i: (i, 0)),
            pl.BlockSpec((tblk, 1), lambda i: (i, 0)),
            pl.BlockSpec((tblk, K * 16), lambda i: (i, 0)),
            pl.BlockSpec((tblk, K), lambda i: (i, 0)),
            pl.BlockSpec((tblk, 1), lambda i: (i, 0)),
        ],
        out_shape=[
            jax.ShapeDtypeStruct((n, DIM), f32),
            jax.ShapeDtypeStruct((N_I, K * 16), f32),
            jax.ShapeDtypeStruct((N_I, K), jnp.int32),
            jax.ShapeDtypeStruct((N_I, 1), f32),
            jax.ShapeDtypeStruct((N_I, K * 16), f32),
            jax.ShapeDtypeStruct((N_I, K), jnp.int32),
            jax.ShapeDtypeStruct((N_I, 1), f32),
        ],
    )(adj, ego0, xn_i, xnt_i, xn_t, xnt_t)


def _scale_emb_body(di_ref, dt_ref, emb_ref, oi_ref, ot_ref):
    e = emb_ref[...]
    oi_ref[...] = di_ref[...] * e
    ot_ref[...] = dt_ref[...] * e


def _scale_emb(d_i, d_t, emb):
    return pl.pallas_call(
        _scale_emb_body,
        out_shape=[
            jax.ShapeDtypeStruct((N_I, DIM), jnp.float32),
            jax.ShapeDtypeStruct((N_I, DIM), jnp.float32),
        ],
    )(d_i, d_t, emb)


_SC_ROWS = 128
_SC_CHUNK = 32
_SC_EDGE = _SC_CHUNK * K
_SC_GB = 64


def _sc_knn_modality(idx_hbm, wrep_hbm, demb_hbm, out_hbm,
                     idx_v, w_v, rows_v, out_v, sem, wid):
    row0 = wid * _SC_ROWS

    def do_chunk(c, _):
        rbase = row0 + c * _SC_CHUNK
        ebase = rbase * K
        pltpu.sync_copy(idx_hbm.at[pl.ds(ebase, _SC_EDGE)], idx_v)
        pltpu.sync_copy(wrep_hbm.at[pl.ds(ebase, _SC_EDGE)], w_v)
        copies = [
            pltpu.async_copy(demb_hbm.at[idx_v.at[pl.ds(b * _SC_GB, _SC_GB)]],
                             rows_v.at[pl.ds(b * _SC_GB, _SC_GB)], sem)
            for b in range(_SC_EDGE // _SC_GB)
        ]
        for cp in copies:
            cp.wait()

        def row_acc(r, _):
            e0 = r * K
            acc = [jnp.zeros((16,), jnp.float32) for _ in range(DIM // 16)]
            for j in range(K):
                wj = w_v[e0 + j, pl.ds(0, 16)]
                for g in range(DIM // 16):
                    acc[g] = acc[g] + wj * rows_v[e0 + j, pl.ds(g * 16, 16)]
            for g in range(DIM // 16):
                out_v[r, pl.ds(g * 16, 16)] = acc[g]
            return ()

        lax.fori_loop(0, _SC_CHUNK, row_acc, ())
        pltpu.sync_copy(out_v, out_hbm.at[pl.ds(rbase, _SC_CHUNK)])
        return ()

    lax.fori_loop(0, _SC_ROWS // _SC_CHUNK, do_chunk, ())


def _sc_knn_body(idx_i, wrep_i, demb_i, idx_t, wrep_t, demb_t,
                 out_i, out_t, idx_v, w_v, rows_v, out_v, sem):
    wid = lax.axis_index("s") * 2 + lax.axis_index("c")
    _sc_knn_modality(idx_i, wrep_i, demb_i, out_i,
                     idx_v, w_v, rows_v, out_v, sem, wid)
    _sc_knn_modality(idx_t, wrep_t, demb_t, out_t,
                     idx_v, w_v, rows_v, out_v, sem, wid)


def _sc_knn(idx_i, wrep_i, demb_i, idx_t, wrep_t, demb_t):
    f32 = jnp.float32
    return pl.kernel(
        _sc_knn_body,
        mesh=plsc.VectorSubcoreMesh(core_axis_name="c", subcore_axis_name="s"),
        out_type=[
            jax.ShapeDtypeStruct((N_I, DIM), f32),
            jax.ShapeDtypeStruct((N_I, DIM), f32),
        ],
        scratch_types=[
            pltpu.VMEM((_SC_EDGE,), jnp.int32),
            pltpu.VMEM((_SC_EDGE, 16), f32),
            pltpu.VMEM((_SC_EDGE, DIM), f32),
            pltpu.VMEM((_SC_CHUNK, DIM), f32),
            pltpu.SemaphoreType.DMA,
        ],
    )(idx_i, wrep_i, demb_i, idx_t, wrep_t, demb_t)


def _l2_body(adj_ref, ego1f_ref, e0_ref, e1_ref, oi_ref, ot_ref, emb_ref,
             mean_ref, mmi_ref, mmt_ref):
    acc = jnp.dot(adj_ref[...], ego1f_ref[...],
                  preferred_element_type=jnp.float32)
    mean_ref[...] = (e0_ref[...] + e1_ref[...] + acc) / 3.0
    emb = emb_ref[...]
    mmi_ref[...] = jnp.dot(oi_ref[...], emb,
                           preferred_element_type=jnp.float32)
    mmt_ref[...] = jnp.dot(ot_ref[...], emb,
                           preferred_element_type=jnp.float32)


def _l2(adj, ego1, ego0, orig_i, orig_t, emb, blk=256):
    n = adj.shape[0]
    grid = (n // blk,)
    oblk = N_I // (n // blk)
    f32 = jnp.float32
    return pl.pallas_call(
        _l2_body,
        grid=grid,
        in_specs=[
            pl.BlockSpec((blk, n), lambda i: (i, 0)),
            pl.BlockSpec((n, DIM), lambda i: (0, 0)),
            pl.BlockSpec((blk, DIM), lambda i: (i, 0)),
            pl.BlockSpec((blk, DIM), lambda i: (i, 0)),
            pl.BlockSpec((oblk, N_I), lambda i: (i, 0)),
            pl.BlockSpec((oblk, N_I), lambda i: (i, 0)),
            pl.BlockSpec((N_I, DIM), lambda i: (0, 0)),
        ],
        out_specs=[
            pl.BlockSpec((blk, DIM), lambda i: (i, 0)),
            pl.BlockSpec((oblk, DIM), lambda i: (i, 0)),
            pl.BlockSpec((oblk, DIM), lambda i: (i, 0)),
        ],
        out_shape=[
            jax.ShapeDtypeStruct((n, DIM), f32),
            jax.ShapeDtypeStruct((N_I, DIM), f32),
            jax.ShapeDtypeStruct((N_I, DIM), f32),
        ],
    )(adj, ego1, ego0, ego1, orig_i, orig_t, emb)


def _fusion_body(mmi_ref, mmt_ref, ki_ref, kt_ref, wq1_ref, bq1_ref,
                 wq2_ref, mi_ref, img_ref, txt_ref, h_ref, ig_ref):
    img = 0.9 * mmi_ref[...] + (1.0 - 0.9) * ki_ref[...]
    txt = 0.9 * mmt_ref[...] + (1.0 - 0.9) * kt_ref[...]
    w1 = wq1_ref[...]
    b1 = bq1_ref[...]
    w2 = wq2_ref[...]
    qi = jnp.dot(jnp.tanh(jnp.dot(img, w1, preferred_element_type=jnp.float32)
                          + b1), w2, preferred_element_type=jnp.float32)
    qt = jnp.dot(jnp.tanh(jnp.dot(txt, w1, preferred_element_type=jnp.float32)
                          + b1), w2, preferred_element_type=jnp.float32)
    m = jnp.maximum(qi, qt)
    ei = jnp.exp(qi - m)
    et = jnp.exp(qt - m)
    s = ei + et
    h = (ei / s) * img + (et / s) * txt
    img_ref[...] = img
    txt_ref[...] = txt
    h_ref[...] = h
    ig_ref[...] = mi_ref[...] + h


def _fusion(mm_img, mm_txt, knn_i, knn_t, wq1, bq1, wq2, mean_items):
    f32 = jnp.float32
    return pl.pallas_call(
        _fusion_body,
        out_shape=[
            jax.ShapeDtypeStruct((N_I, DIM), f32),
            jax.ShapeDtypeStruct((N_I, DIM), f32),
            jax.ShapeDtypeStruct((N_I, DIM), f32),
            jax.ShapeDtypeStruct((N_I, DIM), f32),
        ],
    )(mm_img, mm_txt, knn_i, knn_t, wq1, bq1, wq2, mean_items)


def kernel(adj, feat_visual, feat_text, user_emb, item_emb, W_img, b_img,
           W_txt, b_txt, Wq1, bq1, Wq2, image_original_adj, text_original_adj):
    xn_i = _proj_norm(feat_visual, W_img, b_img.reshape(1, DIM))
    xn_t = _proj_norm(feat_text, W_txt, b_txt.reshape(1, DIM))
    ego0 = jnp.concatenate([user_emb, item_emb], axis=0)
    (ego1, wrep_i, idx_i, d_i,
     wrep_t, idx_t, d_t) = _l1(adj, ego0, xn_i, xn_i.T, xn_t, xn_t.T)
    demb_i, demb_t = _scale_emb(d_i, d_t, item_emb)
    knn_i, knn_t = _sc_knn(idx_i.reshape(-1), wrep_i.reshape(N_I * K, 16),
                           demb_i, idx_t.reshape(-1),
                           wrep_t.reshape(N_I * K, 16), demb_t)
    mean_emb, mm_img, mm_txt = _l2(adj, ego1, ego0, image_original_adj,
                                   text_original_adj, item_emb)
    img_e, txt_e, h, ig = _fusion(mm_img, mm_txt, knn_i, knn_t, Wq1,
                                  bq1.reshape(1, DIM), Wq2, mean_emb[N_U:])
    return (mean_emb[:N_U], ig, img_e, txt_e, h)


# speedup vs baseline: 8.7846x; 1.0161x over previous
import functools

import jax
import jax.numpy as jnp
from jax import lax
from jax.experimental import pallas as pl
from jax.experimental.pallas import tpu as pltpu
from jax.experimental.pallas import tpu_sc as plsc

N_U = 4096
N_I = 4096
DIM = 128
K = 10
NEG = -3.0e38


def _proj_norm_body(feat_ref, w_ref, b_ref, xn_ref):
    y = jnp.dot(feat_ref[...], w_ref[...], preferred_element_type=jnp.float32)
    y = y + b_ref[...]
    n = jnp.sqrt(jnp.sum(y * y, axis=1, keepdims=True))
    xn_ref[...] = y / n


def _proj_norm(feat, w, b):
    return pl.pallas_call(
        _proj_norm_body,
        out_shape=jax.ShapeDtypeStruct((N_I, DIM), jnp.float32),
    )(feat, w, b)


def _topk_compute(xb_ref, xnt_ref, wrep_ref, idx_ref, d_ref):
    a = xb_ref[...]
    sim = jnp.dot(a, xnt_ref[...], preferred_element_type=jnp.float32)
    col = jax.lax.broadcasted_iota(jnp.int32, sim.shape, 1)
    work = sim
    vals = []
    idxs = []
    for _ in range(K):
        m = jnp.max(work, axis=1, keepdims=True)
        ismax = work >= m
        idxs.append(jnp.min(jnp.where(ismax, col, N_I), axis=1,
                            keepdims=True))
        vals.append(m)
        work = jnp.where(ismax, NEG, work)
    v = jnp.concatenate(vals, axis=1)
    ix = jnp.concatenate(idxs, axis=1)
    rowsum = jnp.sum(v, axis=1, keepdims=True)
    dd = jnp.where(rowsum > 0.0,
                   jax.lax.rsqrt(jnp.where(rowsum > 0.0, rowsum, 1.0)), 0.0)
    coeff = v * dd
    wrep_ref[...] = jnp.broadcast_to(coeff[:, :, None],
                                     coeff.shape + (16,)).reshape(
                                         coeff.shape[0], K * 16)
    idx_ref[...] = ix
    d_ref[...] = dd


def _l1_body(adj_ref, ego_ref, xbi_ref, xnti_ref, xbt_ref, xntt_ref,
             oi_ref, emb_ref,
             ego1_ref, wrepi_ref, idxi_ref, di_ref,
             wrept_ref, idxt_ref, dt_ref, mmi_ref):
    ego1_ref[...] = jnp.dot(adj_ref[...], ego_ref[...],
                            preferred_element_type=jnp.float32)
    mmi_ref[...] = jnp.dot(oi_ref[...], emb_ref[...],
                           preferred_element_type=jnp.float32)
    _topk_compute(xbi_ref, xnti_ref, wrepi_ref, idxi_ref, di_ref)
    _topk_compute(xbt_ref, xntt_ref, wrept_ref, idxt_ref, dt_ref)


def _l1(adj, ego0, xn_i, xnt_i, xn_t, xnt_t, orig_i, emb, blk=256):
    n = adj.shape[0]
    grid = (n // blk,)
    tblk = N_I // (n // blk)
    f32 = jnp.float32
    return pl.pallas_call(
        _l1_body,
        grid=grid,
        in_specs=[
            pl.BlockSpec((blk, n), lambda i: (i, 0)),
            pl.BlockSpec((n, DIM), lambda i: (0, 0)),
            pl.BlockSpec((tblk, DIM), lambda i: (i, 0)),
            pl.BlockSpec((DIM, N_I), lambda i: (0, 0)),
            pl.BlockSpec((tblk, DIM), lambda i: (i, 0)),
            pl.BlockSpec((DIM, N_I), lambda i: (0, 0)),
            pl.BlockSpec((tblk, N_I), lambda i: (i, 0)),
            pl.BlockSpec((N_I, DIM), lambda i: (0, 0)),
        ],
        out_specs=[
            pl.BlockSpec((blk, DIM), lambda i: (i, 0)),
            pl.BlockSpec((tblk, K * 16), lambda i: (i, 0)),
            pl.BlockSpec((tblk, K), lambda i: (i, 0)),
            pl.BlockSpec((tblk, 1), lambda i: (i, 0)),
            pl.BlockSpec((tblk, K * 16), lambda i: (i, 0)),
            pl.BlockSpec((tblk, K), lambda i: (i, 0)),
            pl.BlockSpec((tblk, 1), lambda i: (i, 0)),
            pl.BlockSpec((tblk, DIM), lambda i: (i, 0)),
        ],
        out_shape=[
            jax.ShapeDtypeStruct((n, DIM), f32),
            jax.ShapeDtypeStruct((N_I, K * 16), f32),
            jax.ShapeDtypeStruct((N_I, K), jnp.int32),
            jax.ShapeDtypeStruct((N_I, 1), f32),
            jax.ShapeDtypeStruct((N_I, K * 16), f32),
            jax.ShapeDtypeStruct((N_I, K), jnp.int32),
            jax.ShapeDtypeStruct((N_I, 1), f32),
            jax.ShapeDtypeStruct((N_I, DIM), f32),
        ],
    )(adj, ego0, xn_i, xnt_i, xn_t, xnt_t, orig_i, emb)


def _scale_emb_body(di_ref, dt_ref, emb_ref, oi_ref, ot_ref):
    e = emb_ref[...]
    oi_ref[...] = di_ref[...] * e
    ot_ref[...] = dt_ref[...] * e


def _scale_emb(d_i, d_t, emb):
    return pl.pallas_call(
        _scale_emb_body,
        out_shape=[
            jax.ShapeDtypeStruct((N_I, DIM), jnp.float32),
            jax.ShapeDtypeStruct((N_I, DIM), jnp.float32),
        ],
    )(d_i, d_t, emb)


_SC_ROWS = 128
_SC_CHUNK = 32
_SC_EDGE = _SC_CHUNK * K
_SC_GB = 64


def _sc_knn_modality(idx_hbm, wrep_hbm, demb_hbm, out_hbm,
                     idx_v, w_v, rows_v, out_v, sem, wid):
    row0 = wid * _SC_ROWS

    def do_chunk(c, _):
        rbase = row0 + c * _SC_CHUNK
        ebase = rbase * K
        pltpu.sync_copy(idx_hbm.at[pl.ds(ebase, _SC_EDGE)], idx_v)
        pltpu.sync_copy(wrep_hbm.at[pl.ds(ebase, _SC_EDGE)], w_v)
        copies = [
            pltpu.async_copy(demb_hbm.at[idx_v.at[pl.ds(b * _SC_GB, _SC_GB)]],
                             rows_v.at[pl.ds(b * _SC_GB, _SC_GB)], sem)
            for b in range(_SC_EDGE // _SC_GB)
        ]
        for cp in copies:
            cp.wait()

        def row_acc(r, _):
            e0 = r * K
            acc = [jnp.zeros((16,), jnp.float32) for _ in range(DIM // 16)]
            for j in range(K):
                wj = w_v[e0 + j, pl.ds(0, 16)]
                for g in range(DIM // 16):
                    acc[g] = acc[g] + wj * rows_v[e0 + j, pl.ds(g * 16, 16)]
            for g in range(DIM // 16):
                out_v[r, pl.ds(g * 16, 16)] = acc[g]
            return ()

        lax.fori_loop(0, _SC_CHUNK, row_acc, ())
        pltpu.sync_copy(out_v, out_hbm.at[pl.ds(rbase, _SC_CHUNK)])
        return ()

    lax.fori_loop(0, _SC_ROWS // _SC_CHUNK, do_chunk, ())


def _sc_knn_body(idx_i, wrep_i, demb_i, idx_t, wrep_t, demb_t,
                 out_i, out_t, idx_v, w_v, rows_v, out_v, sem):
    wid = lax.axis_index("s") * 2 + lax.axis_index("c")
    _sc_knn_modality(idx_i, wrep_i, demb_i, out_i,
                     idx_v, w_v, rows_v, out_v, sem, wid)
    _sc_knn_modality(idx_t, wrep_t, demb_t, out_t,
                     idx_v, w_v, rows_v, out_v, sem, wid)


def _sc_knn(idx_i, wrep_i, demb_i, idx_t, wrep_t, demb_t):
    f32 = jnp.float32
    return pl.kernel(
        _sc_knn_body,
        mesh=plsc.VectorSubcoreMesh(core_axis_name="c", subcore_axis_name="s"),
        out_type=[
            jax.ShapeDtypeStruct((N_I, DIM), f32),
            jax.ShapeDtypeStruct((N_I, DIM), f32),
        ],
        scratch_types=[
            pltpu.VMEM((_SC_EDGE,), jnp.int32),
            pltpu.VMEM((_SC_EDGE, 16), f32),
            pltpu.VMEM((_SC_EDGE, DIM), f32),
            pltpu.VMEM((_SC_CHUNK, DIM), f32),
            pltpu.SemaphoreType.DMA,
        ],
    )(idx_i, wrep_i, demb_i, idx_t, wrep_t, demb_t)


def _l2_body(adj_ref, ego1f_ref, e0_ref, e1_ref, ot_ref, emb_ref,
             mean_ref, mmt_ref):
    acc = jnp.dot(adj_ref[...], ego1f_ref[...],
                  preferred_element_type=jnp.float32)
    mean_ref[...] = (e0_ref[...] + e1_ref[...] + acc) / 3.0
    mmt_ref[...] = jnp.dot(ot_ref[...], emb_ref[...],
                           preferred_element_type=jnp.float32)


def _l2(adj, ego1, ego0, orig_t, emb, blk=256):
    n = adj.shape[0]
    grid = (n // blk,)
    oblk = N_I // (n // blk)
    f32 = jnp.float32
    return pl.pallas_call(
        _l2_body,
        grid=grid,
        in_specs=[
            pl.BlockSpec((blk, n), lambda i: (i, 0)),
            pl.BlockSpec((n, DIM), lambda i: (0, 0)),
            pl.BlockSpec((blk, DIM), lambda i: (i, 0)),
            pl.BlockSpec((blk, DIM), lambda i: (i, 0)),
            pl.BlockSpec((oblk, N_I), lambda i: (i, 0)),
            pl.BlockSpec((N_I, DIM), lambda i: (0, 0)),
        ],
        out_specs=[
            pl.BlockSpec((blk, DIM), lambda i: (i, 0)),
            pl.BlockSpec((oblk, DIM), lambda i: (i, 0)),
        ],
        out_shape=[
            jax.ShapeDtypeStruct((n, DIM), f32),
            jax.ShapeDtypeStruct((N_I, DIM), f32),
        ],
    )(adj, ego1, ego0, ego1, orig_t, emb)


def _fusion_body(mmi_ref, mmt_ref, ki_ref, kt_ref, wq1_ref, bq1_ref,
                 wq2_ref, mi_ref, img_ref, txt_ref, h_ref, ig_ref):
    img = 0.9 * mmi_ref[...] + (1.0 - 0.9) * ki_ref[...]
    txt = 0.9 * mmt_ref[...] + (1.0 - 0.9) * kt_ref[...]
    w1 = wq1_ref[...]
    b1 = bq1_ref[...]
    w2 = wq2_ref[...]
    qi = jnp.dot(jnp.tanh(jnp.dot(img, w1, preferred_element_type=jnp.float32)
                          + b1), w2, preferred_element_type=jnp.float32)
    qt = jnp.dot(jnp.tanh(jnp.dot(txt, w1, preferred_element_type=jnp.float32)
                          + b1), w2, preferred_element_type=jnp.float32)
    m = jnp.maximum(qi, qt)
    ei = jnp.exp(qi - m)
    et = jnp.exp(qt - m)
    s = ei + et
    h = (ei / s) * img + (et / s) * txt
    img_ref[...] = img
    txt_ref[...] = txt
    h_ref[...] = h
    ig_ref[...] = mi_ref[...] + h


def _fusion(mm_img, mm_txt, knn_i, knn_t, wq1, bq1, wq2, mean_items):
    f32 = jnp.float32
    return pl.pallas_call(
        _fusion_body,
        out_shape=[
            jax.ShapeDtypeStruct((N_I, DIM), f32),
            jax.ShapeDtypeStruct((N_I, DIM), f32),
            jax.ShapeDtypeStruct((N_I, DIM), f32),
            jax.ShapeDtypeStruct((N_I, DIM), f32),
        ],
    )(mm_img, mm_txt, knn_i, knn_t, wq1, bq1, wq2, mean_items)


def kernel(adj, feat_visual, feat_text, user_emb, item_emb, W_img, b_img,
           W_txt, b_txt, Wq1, bq1, Wq2, image_original_adj, text_original_adj):
    xn_i = _proj_norm(feat_visual, W_img, b_img.reshape(1, DIM))
    xn_t = _proj_norm(feat_text, W_txt, b_txt.reshape(1, DIM))
    ego0 = jnp.concatenate([user_emb, item_emb], axis=0)
    (ego1, wrep_i, idx_i, d_i, wrep_t, idx_t, d_t,
     mm_img) = _l1(adj, ego0, xn_i, xn_i.T, xn_t, xn_t.T,
                   image_original_adj, item_emb)
    demb_i, demb_t = _scale_emb(d_i, d_t, item_emb)
    knn_i, knn_t = _sc_knn(idx_i.reshape(-1), wrep_i.reshape(N_I * K, 16),
                           demb_i, idx_t.reshape(-1),
                           wrep_t.reshape(N_I * K, 16), demb_t)
    mean_emb, mm_txt = _l2(adj, ego1, ego0, text_original_adj, item_emb)
    img_e, txt_e, h, ig = _fusion(mm_img, mm_txt, knn_i, knn_t, Wq1,
                                  bq1.reshape(1, DIM), Wq2, mean_emb[N_U:])
    return (mean_emb[:N_U], ig, img_e, txt_e, h)
